# initial kernel scaffold (unmeasured)
import jax
import jax.numpy as jnp
from jax import lax
from jax.experimental import pallas as pl
from jax.experimental.pallas import tpu as pltpu


def kernel(O, Wo):
    B, S, Hl, D = O.shape
    K = Hl * D
    N = Wo.shape[1]
    s_half = S // 2

    x = lax.axis_index("x")
    A = O.reshape(B, S, K).astype(jnp.bfloat16)
    W = Wo.astype(jnp.bfloat16)
    A_send = lax.dynamic_slice_in_dim(A, (1 - x) * s_half, s_half, axis=1)
    A_keep = lax.dynamic_slice_in_dim(A, x * s_half, s_half, axis=1)

    def body(a_send_ref, a_keep_ref, w_ref, out_ref,
             send_buf, recv_buf, send_sems, recv_sems):
        my_x = lax.axis_index("x")
        my_y = lax.axis_index("y")
        my_z = lax.axis_index("z")
        partner = (1 - my_x, my_y, my_z)

        barrier = pltpu.get_barrier_semaphore()
        pl.semaphore_signal(
            barrier, inc=1,
            device_id=partner, device_id_type=pl.DeviceIdType.MESH,
        )
        pl.semaphore_wait(barrier, 1)

        w = w_ref[...]

        rdmas = []
        for b in range(B):
            send_buf[b] = jnp.dot(
                a_send_ref[b], w, preferred_element_type=jnp.float32
            ).astype(jnp.bfloat16)
            rdma = pltpu.make_async_remote_copy(
                src_ref=send_buf.at[b],
                dst_ref=recv_buf.at[b],
                send_sem=send_sems.at[b],
                recv_sem=recv_sems.at[b],
                device_id=partner,
                device_id_type=pl.DeviceIdType.MESH,
            )
            rdma.start()
            rdmas.append(rdma)

        for b in range(B):
            out_ref[b] = jnp.dot(
                a_keep_ref[b], w, preferred_element_type=jnp.float32
            )

        for b in range(B):
            rdmas[b].wait_send()
            rdmas[b].wait_recv()
            out_ref[b] = out_ref[b] + recv_buf[b].astype(jnp.float32)

    return pl.pallas_call(
        body,
        out_shape=jax.ShapeDtypeStruct((B, s_half, N), jnp.float32),
        in_specs=[
            pl.BlockSpec(memory_space=pltpu.VMEM),
            pl.BlockSpec(memory_space=pltpu.VMEM),
            pl.BlockSpec(memory_space=pltpu.VMEM),
        ],
        out_specs=pl.BlockSpec(memory_space=pltpu.VMEM),
        scratch_shapes=[
            pltpu.VMEM((B, s_half, N), jnp.bfloat16),
            pltpu.VMEM((B, s_half, N), jnp.bfloat16),
            pltpu.SemaphoreType.DMA((B,)),
            pltpu.SemaphoreType.DMA((B,)),
        ],
        compiler_params=pltpu.CompilerParams(collective_id=0),
    )(A_send, A_keep, W)


# baseline (device time: 330069 ns/iter reference)
import jax
import jax.numpy as jnp
from jax import lax
from jax.experimental import pallas as pl
from jax.experimental.pallas import tpu as pltpu

N_SLOTS = 2


def kernel(O, Wo):
    B, S, Hl, D = O.shape
    K = Hl * D
    N = Wo.shape[1]
    s_half = S // 2
    n_chunks = 2
    n_blk = N // n_chunks
    T = n_chunks * B

    x = lax.axis_index("x")
    A = O.reshape(B, S, K).astype(jnp.bfloat16)
    W = Wo.astype(jnp.bfloat16)
    A_send = lax.dynamic_slice_in_dim(A, (1 - x) * s_half, s_half, axis=1)
    A_keep = lax.dynamic_slice_in_dim(A, x * s_half, s_half, axis=1)

    def body(a_send_ref, a_keep_ref, w_ref, out_ref,
             send_buf, recv_buf, send_sems, recv_sems, credit_sem):
        n = pl.program_id(0)
        b = pl.program_id(1)
        t = n * B + b
        slot = t % N_SLOTS

        my_x = lax.axis_index("x")
        my_y = lax.axis_index("y")
        my_z = lax.axis_index("z")
        partner = (1 - my_x, my_y, my_z)

        def rdma_for(s):
            return pltpu.make_async_remote_copy(
                src_ref=send_buf.at[s],
                dst_ref=recv_buf.at[s],
                send_sem=send_sems.at[s],
                recv_sem=recv_sems.at[s],
                device_id=partner,
                device_id_type=pl.DeviceIdType.MESH,
            )

        @pl.when(t == 0)
        def _():
            barrier = pltpu.get_barrier_semaphore()
            pl.semaphore_signal(
                barrier, inc=1,
                device_id=partner, device_id_type=pl.DeviceIdType.MESH,
            )
            pl.semaphore_wait(barrier, 1)

        @pl.when(t >= N_SLOTS)
        def _():
            rdma_for(slot).wait_send()
            pl.semaphore_wait(credit_sem, 1)

        send_buf[slot] = jnp.dot(
            a_send_ref[0], w_ref[...], preferred_element_type=jnp.float32
        ).astype(jnp.bfloat16)
        rdma = rdma_for(slot)
        rdma.start()

        out_ref[0] = jnp.dot(
            a_keep_ref[0], w_ref[...], preferred_element_type=jnp.float32
        )

        rdma.wait_recv()
        out_ref[0] = out_ref[0] + recv_buf[slot].astype(jnp.float32)

        @pl.when(t < T - N_SLOTS)
        def _():
            pl.semaphore_signal(
                credit_sem, inc=1,
                device_id=partner, device_id_type=pl.DeviceIdType.MESH,
            )

        @pl.when(t == T - 1)
        def _():
            for s in range(N_SLOTS):
                rdma_for(s).wait_send()

    return pl.pallas_call(
        body,
        grid=(n_chunks, B),
        out_shape=jax.ShapeDtypeStruct((B, s_half, N), jnp.float32),
        in_specs=[
            pl.BlockSpec((1, s_half, K), lambda n, b: (b, 0, 0),
                         memory_space=pltpu.VMEM),
            pl.BlockSpec((1, s_half, K), lambda n, b: (b, 0, 0),
                         memory_space=pltpu.VMEM),
            pl.BlockSpec((K, n_blk), lambda n, b: (0, n),
                         memory_space=pltpu.VMEM),
        ],
        out_specs=pl.BlockSpec((1, s_half, n_blk), lambda n, b: (b, 0, n),
                               memory_space=pltpu.VMEM),
        scratch_shapes=[
            pltpu.VMEM((N_SLOTS, s_half, n_blk), jnp.bfloat16),
            pltpu.VMEM((N_SLOTS, s_half, n_blk), jnp.bfloat16),
            pltpu.SemaphoreType.DMA((N_SLOTS,)),
            pltpu.SemaphoreType.DMA((N_SLOTS,)),
            pltpu.SemaphoreType.REGULAR,
        ],
        compiler_params=pltpu.CompilerParams(
            collective_id=0,
            dimension_semantics=("arbitrary", "arbitrary"),
            vmem_limit_bytes=60 * 1024 * 1024,
        ),
    )(A_send, A_keep, W)


# device time: 279346 ns/iter; 1.1816x vs baseline; 1.1816x over previous
import jax
import jax.numpy as jnp
from jax import lax
from jax.experimental import pallas as pl
from jax.experimental.pallas import tpu as pltpu

N_SLOTS = 2


def kernel(O, Wo):
    B, S, Hl, D = O.shape
    K = Hl * D
    N = Wo.shape[1]
    s_half = S // 2
    n_chunks = 2
    n_blk = N // n_chunks
    T = n_chunks * B

    x = lax.axis_index("x")
    A = O.reshape(B, S, K).astype(jnp.bfloat16)
    W = Wo.astype(jnp.bfloat16)
    A_send = lax.dynamic_slice_in_dim(A, (1 - x) * s_half, s_half, axis=1)
    A_keep = lax.dynamic_slice_in_dim(A, x * s_half, s_half, axis=1)

    def body(a_send_ref, a_keep_ref, w_ref, out_ref,
             send_buf, recv_buf, keep_buf, send_sems, recv_sems, credit_sem):
        t = pl.program_id(0)
        slot = t % N_SLOTS
        prev_slot = (t + 1) % N_SLOTS

        my_x = lax.axis_index("x")
        my_y = lax.axis_index("y")
        my_z = lax.axis_index("z")
        partner = (1 - my_x, my_y, my_z)

        def rdma_for(s):
            return pltpu.make_async_remote_copy(
                src_ref=send_buf.at[s],
                dst_ref=recv_buf.at[s],
                send_sem=send_sems.at[s],
                recv_sem=recv_sems.at[s],
                device_id=partner,
                device_id_type=pl.DeviceIdType.MESH,
            )

        @pl.when(t == 0)
        def _():
            barrier = pltpu.get_barrier_semaphore()
            pl.semaphore_signal(
                barrier, inc=1,
                device_id=partner, device_id_type=pl.DeviceIdType.MESH,
            )
            pl.semaphore_wait(barrier, 1)

        @pl.when(jnp.logical_and(t >= N_SLOTS, t < T))
        def _():
            rdma_for(slot).wait_send()
            pl.semaphore_wait(credit_sem, 1)

        @pl.when(t < T)
        def _():
            send_buf[slot] = jnp.dot(
                a_send_ref[0], w_ref[...], preferred_element_type=jnp.float32
            ).astype(jnp.bfloat16)
            rdma_for(slot).start()
            keep_buf[slot] = jnp.dot(
                a_keep_ref[0], w_ref[...], preferred_element_type=jnp.float32
            )

        @pl.when(t > 0)
        def _():
            rdma_for(prev_slot).wait_recv()
            out_ref[0] = keep_buf[prev_slot] + recv_buf[prev_slot].astype(
                jnp.float32
            )

        @pl.when(jnp.logical_and(t > 0, t - 1 < T - N_SLOTS))
        def _():
            pl.semaphore_signal(
                credit_sem, inc=1,
                device_id=partner, device_id_type=pl.DeviceIdType.MESH,
            )

        @pl.when(t == T)
        def _():
            for s in range(N_SLOTS):
                rdma_for(s).wait_send()

    def a_map(t):
        te = jnp.minimum(t, T - 1)
        return (te % B, 0, 0)

    def w_map(t):
        te = jnp.minimum(t, T - 1)
        return (0, te // B)

    def out_map(t):
        to = jnp.maximum(t - 1, 0)
        return (to % B, 0, to // B)

    return pl.pallas_call(
        body,
        grid=(T + 1,),
        out_shape=jax.ShapeDtypeStruct((B, s_half, N), jnp.float32),
        in_specs=[
            pl.BlockSpec((1, s_half, K), a_map, memory_space=pltpu.VMEM),
            pl.BlockSpec((1, s_half, K), a_map, memory_space=pltpu.VMEM),
            pl.BlockSpec((K, n_blk), w_map, memory_space=pltpu.VMEM),
        ],
        out_specs=pl.BlockSpec((1, s_half, n_blk), out_map,
                               memory_space=pltpu.VMEM),
        scratch_shapes=[
            pltpu.VMEM((N_SLOTS, s_half, n_blk), jnp.bfloat16),
            pltpu.VMEM((N_SLOTS, s_half, n_blk), jnp.bfloat16),
            pltpu.VMEM((N_SLOTS, s_half, n_blk), jnp.float32),
            pltpu.SemaphoreType.DMA((N_SLOTS,)),
            pltpu.SemaphoreType.DMA((N_SLOTS,)),
            pltpu.SemaphoreType.REGULAR,
        ],
        compiler_params=pltpu.CompilerParams(
            collective_id=0,
            dimension_semantics=("arbitrary",),
            vmem_limit_bytes=60 * 1024 * 1024,
        ),
    )(A_send, A_keep, W)


# device time: 264767 ns/iter; 1.2466x vs baseline; 1.0551x over previous
import jax
import jax.numpy as jnp
from jax import lax
from jax.experimental import pallas as pl
from jax.experimental.pallas import tpu as pltpu

N_SLOTS = 2


def kernel(O, Wo):
    B, S, Hl, D = O.shape
    K = Hl * D
    N = Wo.shape[1]
    s_half = S // 2
    n_chunks = 2
    n_blk = N // n_chunks
    T = n_chunks * B

    A = O.reshape(B, 2, s_half, K).astype(jnp.bfloat16)
    W = Wo.astype(jnp.bfloat16)

    def body(a_ref, w_ref, out_ref,
             send_buf, recv_buf, keep_buf, send_sems, recv_sems, credit_sem):
        t = pl.program_id(0)
        slot = t % N_SLOTS
        prev_slot = (t + 1) % N_SLOTS

        my_x = lax.axis_index("x")
        my_y = lax.axis_index("y")
        my_z = lax.axis_index("z")
        partner = (1 - my_x, my_y, my_z)

        def rdma_for(s):
            return pltpu.make_async_remote_copy(
                src_ref=send_buf.at[s],
                dst_ref=recv_buf.at[s],
                send_sem=send_sems.at[s],
                recv_sem=recv_sems.at[s],
                device_id=partner,
                device_id_type=pl.DeviceIdType.MESH,
            )

        @pl.when(t == 0)
        def _():
            barrier = pltpu.get_barrier_semaphore()
            pl.semaphore_signal(
                barrier, inc=1,
                device_id=partner, device_id_type=pl.DeviceIdType.MESH,
            )
            pl.semaphore_wait(barrier, 1)

        @pl.when(jnp.logical_and(t >= N_SLOTS, t < T))
        def _():
            rdma_for(slot).wait_send()
            pl.semaphore_wait(credit_sem, 1)

        @pl.when(t < T)
        def _():
            send_buf[slot] = jnp.dot(
                a_ref[0, 1 - my_x], w_ref[...],
                preferred_element_type=jnp.float32,
            ).astype(jnp.bfloat16)
            rdma_for(slot).start()
            keep_buf[slot] = jnp.dot(
                a_ref[0, my_x], w_ref[...],
                preferred_element_type=jnp.float32,
            )

        @pl.when(t > 0)
        def _():
            rdma_for(prev_slot).wait_recv()
            out_ref[0] = keep_buf[prev_slot] + recv_buf[prev_slot].astype(
                jnp.float32
            )

        @pl.when(jnp.logical_and(t > 0, t - 1 < T - N_SLOTS))
        def _():
            pl.semaphore_signal(
                credit_sem, inc=1,
                device_id=partner, device_id_type=pl.DeviceIdType.MESH,
            )

        @pl.when(t == T)
        def _():
            for s in range(N_SLOTS):
                rdma_for(s).wait_send()

    def a_map(t):
        te = jnp.minimum(t, T - 1)
        return (te % B, 0, 0, 0)

    def w_map(t):
        te = jnp.minimum(t, T - 1)
        return (0, te // B)

    def out_map(t):
        to = jnp.maximum(t - 1, 0)
        return (to % B, 0, to // B)

    return pl.pallas_call(
        body,
        grid=(T + 1,),
        out_shape=jax.ShapeDtypeStruct((B, s_half, N), jnp.float32),
        in_specs=[
            pl.BlockSpec((1, 2, s_half, K), a_map, memory_space=pltpu.VMEM),
            pl.BlockSpec((K, n_blk), w_map, memory_space=pltpu.VMEM),
        ],
        out_specs=pl.BlockSpec((1, s_half, n_blk), out_map,
                               memory_space=pltpu.VMEM),
        scratch_shapes=[
            pltpu.VMEM((N_SLOTS, s_half, n_blk), jnp.bfloat16),
            pltpu.VMEM((N_SLOTS, s_half, n_blk), jnp.bfloat16),
            pltpu.VMEM((N_SLOTS, s_half, n_blk), jnp.float32),
            pltpu.SemaphoreType.DMA((N_SLOTS,)),
            pltpu.SemaphoreType.DMA((N_SLOTS,)),
            pltpu.SemaphoreType.REGULAR,
        ],
        compiler_params=pltpu.CompilerParams(
            collective_id=0,
            dimension_semantics=("arbitrary",),
            vmem_limit_bytes=60 * 1024 * 1024,
        ),
    )(A, W)
